# Initial kernel scaffold; baseline (speedup 1.0000x reference)
import jax
import jax.numpy as jnp
from jax.experimental import pallas as pl


def kernel(x, table):
    raise NotImplementedError("write your pallas kernel here")


# speedup vs baseline: 9.2477x; 9.2477x over previous
import functools

import jax
import jax.numpy as jnp
from jax import lax
from jax.experimental import pallas as pl
from jax.experimental.pallas import tpu as pltpu
from jax.experimental.pallas import tpu_sc as plsc

_DIM = 128
_CHUNK = 128
_NBUF = 4


@functools.lru_cache(maxsize=None)
def _build(n_rows: int, vocab: int, dim: int):
    NC, NS = 2, 16
    NW = NC * NS
    assert n_rows % (NW * _CHUNK) == 0
    n_chunks = n_rows // (NW * _CHUNK)
    LOOK = _NBUF // 2

    mesh = plsc.VectorSubcoreMesh(
        core_axis_name="c", subcore_axis_name="s",
        num_cores=NC, num_subcores=NS)

    @functools.partial(
        pl.kernel,
        out_type=jax.ShapeDtypeStruct((NW, n_chunks, _CHUNK, dim), jnp.float32),
        mesh=mesh,
        scratch_types=(
            [pltpu.VMEM((n_chunks, _CHUNK), jnp.int32)]
            + [pltpu.VMEM((_CHUNK, dim), jnp.float32) for _ in range(_NBUF)]
            + [pltpu.SemaphoreType.DMA for _ in range(2 * _NBUF)]
        ),
    )
    def gather_kernel(x_hbm, table_hbm, out_hbm, idx_v, *rest):
        rows = rest[:_NBUF]
        gsem = rest[_NBUF:2 * _NBUF]
        ssem = rest[2 * _NBUF:3 * _NBUF]
        wid = lax.axis_index("s") * NC + lax.axis_index("c")

        pltpu.sync_copy(x_hbm.at[wid], idx_v)

        def start_gather(j, b):
            pltpu.async_copy(table_hbm.at[idx_v.at[j]], rows[b], gsem[b])

        def wait_gather(j, b):
            pltpu.make_async_copy(
                table_hbm.at[idx_v.at[j]], rows[b], gsem[b]).wait()

        def start_store(j, b):
            pltpu.async_copy(rows[b], out_hbm.at[wid, j], ssem[b])

        def wait_store(j, b):
            pltpu.make_async_copy(rows[b], out_hbm.at[wid, j], ssem[b]).wait()

        for b in range(LOOK):
            start_gather(b, b)

        @pl.loop(0, n_chunks, step=_NBUF)
        def _(j0):
            for b in range(_NBUF):
                j = j0 + b
                jn = j + LOOK
                bf = (b + LOOK) % _NBUF

                @pl.when(jnp.logical_and(jn < n_chunks, j >= LOOK))
                def _():
                    wait_store(jn - _NBUF, bf)

                @pl.when(jn < n_chunks)
                def _():
                    start_gather(jn, bf)

                wait_gather(j, b)
                start_store(j, b)

        for b in range(_NBUF):
            j = n_chunks - _NBUF + b
            wait_store(j, b)

    return gather_kernel


def kernel(x, table):
    n_rows = x.shape[0] * x.shape[1]
    vocab, dim = table.shape
    NW = 32
    n_chunks = n_rows // (NW * _CHUNK)
    x3 = jnp.reshape(x, (NW, n_chunks, _CHUNK))
    out = _build(n_rows, vocab, dim)(x3, table)
    return jnp.reshape(out, (x.shape[0], x.shape[1], dim))


# speedup vs baseline: 9.2827x; 1.0038x over previous
import functools

import jax
import jax.numpy as jnp
from jax import lax
from jax.experimental import pallas as pl
from jax.experimental.pallas import tpu as pltpu
from jax.experimental.pallas import tpu_sc as plsc

_DIM = 128
_CHUNK = 128
_NBUF = 5
_LOOK = 3


@functools.lru_cache(maxsize=None)
def _build(n_rows: int, vocab: int, dim: int):
    NC, NS = 2, 16
    NW = NC * NS
    assert n_rows % (NW * _CHUNK) == 0
    n_chunks = n_rows // (NW * _CHUNK)
    assert n_chunks % _NBUF == 0
    LOOK = _LOOK

    mesh = plsc.VectorSubcoreMesh(
        core_axis_name="c", subcore_axis_name="s",
        num_cores=NC, num_subcores=NS)

    @functools.partial(
        pl.kernel,
        out_type=jax.ShapeDtypeStruct((NW, n_chunks, _CHUNK, dim), jnp.float32),
        mesh=mesh,
        scratch_types=(
            [pltpu.VMEM((n_chunks, _CHUNK), jnp.int32)]
            + [pltpu.VMEM((_CHUNK, dim), jnp.float32) for _ in range(_NBUF)]
            + [pltpu.SemaphoreType.DMA for _ in range(2 * _NBUF)]
        ),
    )
    def gather_kernel(x_hbm, table_hbm, out_hbm, idx_v, *rest):
        rows = rest[:_NBUF]
        gsem = rest[_NBUF:2 * _NBUF]
        ssem = rest[2 * _NBUF:3 * _NBUF]
        wid = lax.axis_index("s") * NC + lax.axis_index("c")

        pltpu.sync_copy(x_hbm.at[wid], idx_v)

        def start_gather(j, b):
            pltpu.async_copy(table_hbm.at[idx_v.at[j]], rows[b], gsem[b])

        def wait_gather(j, b):
            pltpu.make_async_copy(
                table_hbm.at[idx_v.at[j]], rows[b], gsem[b]).wait()

        def start_store(j, b):
            pltpu.async_copy(rows[b], out_hbm.at[wid, j], ssem[b])

        def wait_store(j, b):
            pltpu.make_async_copy(rows[b], out_hbm.at[wid, j], ssem[b]).wait()

        for b in range(LOOK):
            start_gather(b, b)

        @pl.loop(0, n_chunks, step=_NBUF)
        def _(j0):
            for b in range(_NBUF):
                j = j0 + b
                jn = j + LOOK
                bf = (b + LOOK) % _NBUF

                @pl.when(jnp.logical_and(jn < n_chunks, j >= _NBUF - LOOK))
                def _():
                    wait_store(jn - _NBUF, bf)

                @pl.when(jn < n_chunks)
                def _():
                    start_gather(jn, bf)

                wait_gather(j, b)
                start_store(j, b)

        for b in range(_NBUF):
            j = n_chunks - _NBUF + b
            wait_store(j, b)

    return gather_kernel


def kernel(x, table):
    n_rows = x.shape[0] * x.shape[1]
    vocab, dim = table.shape
    NW = 32
    n_chunks = n_rows // (NW * _CHUNK)
    x3 = jnp.reshape(x, (NW, n_chunks, _CHUNK))
    out = _build(n_rows, vocab, dim)(x3, table)
    return jnp.reshape(out, (x.shape[0], x.shape[1], dim))
